# baseline (speedup 1.0000x reference)
import functools

import jax
import jax.numpy as jnp
import numpy as np
from jax.experimental import pallas as pl

_D_IN = 1025
_EPS_BN = 1e-5


def _mm_body(a_ref, b_ref, o_ref):
    o_ref[...] = jnp.dot(a_ref[...], b_ref[...],
                         preferred_element_type=jnp.float32)


def _pallas_mm(a, b, block_m=400):
    m, k = a.shape
    k2, n = b.shape
    assert k == k2 and m % block_m == 0
    grid = (m // block_m,)
    return pl.pallas_call(
        _mm_body,
        grid=grid,
        in_specs=[
            pl.BlockSpec((block_m, k), lambda i: (i, 0)),
            pl.BlockSpec((k, n), lambda i: (0, 0)),
        ],
        out_specs=pl.BlockSpec((block_m, n), lambda i: (i, 0)),
        out_shape=jax.ShapeDtypeStruct((m, n), jnp.float32),
    )(a, b)


def _seg_softmax(e, idx, n):
    m = jax.ops.segment_max(e, idx, num_segments=n)
    m = jnp.where(jnp.isfinite(m), m, 0.0)
    ex = jnp.exp(e - m[idx])
    s = jax.ops.segment_sum(ex, idx, num_segments=n)
    return ex / (s[idx] + 1e-16)


def _greedy_merge(e, src, dst, num_nodes):
    perm = jnp.argsort(-e)

    def body(k, state):
        mask, cluster, score, i = state
        eidx = perm[k]
        s = src[eidx]
        t = dst[eidx]
        ok = mask[s] & mask[t]
        cluster = cluster.at[s].set(jnp.where(ok, i, cluster[s]))
        cluster = cluster.at[t].set(jnp.where(ok, i, cluster[t]))
        score = score.at[i].set(jnp.where(ok, e[eidx], score[i]))
        mask = mask.at[s].set(mask[s] & ~ok)
        mask = mask.at[t].set(mask[t] & ~ok)
        return mask, cluster, score, i + ok.astype(jnp.int32)

    mask0 = jnp.ones((num_nodes,), dtype=bool)
    cluster0 = jnp.zeros((num_nodes,), dtype=jnp.int32)
    score0 = jnp.ones((num_nodes,), dtype=e.dtype)
    mask, cluster, score, i = jax.lax.fori_loop(
        0, e.shape[0], body, (mask0, cluster0, score0, jnp.int32(0)))
    csum = jnp.cumsum(mask.astype(jnp.int32))
    cluster = jnp.where(mask, i + csum - 1, cluster)
    n_new = i + csum[-1]
    return cluster, score, i, n_new


def _cheb(x, src, dst, W0, W1, bias, n):
    w = jnp.where(src == dst, 0.0, 1.0).astype(x.dtype)
    deg = jax.ops.segment_sum(w, src, num_segments=n)
    dis = jnp.where(deg > 0, 1.0 / jnp.sqrt(jnp.maximum(deg, 1e-12)), 0.0)
    norm = -dis[src] * w * dis[dst]
    xw1 = _pallas_mm(x, W1)
    tx1 = jax.ops.segment_sum(norm[:, None] * xw1[src], dst, num_segments=n)
    return _pallas_mm(x, W0) + tx1 + bias


def _bn(h, g, b):
    return h * (g / np.sqrt(1.0 + _EPS_BN)) + b


def kernel(x, edge_index, batch, A, pool_W, pool_b, c1_W0, c1_W1, c1_b,
           bn1_g, bn1_b, c2_W0, c2_W1, c2_b, bn2_g, bn2_b, l1_W, l1_b,
           bn3_g, bn3_b, l2_W, l2_b, bn4_g, bn4_b, l3_W, l3_b):
    src, dst = edge_index[0], edge_index[1]
    n = x.shape[0]
    ab = _pallas_mm(x, jnp.concatenate(
        [pool_W[:_D_IN, :], pool_W[_D_IN:, :]], axis=1))
    raw = ab[src, 0] + ab[dst, 1] + pool_b[0]
    e = _seg_softmax(raw, dst, n) + 0.5
    cluster_j, score, n_merged, n_new = _greedy_merge(e, src, dst, n)
    pair = cluster_j[src] * n_new + cluster_j[dst]
    sp = jnp.sort(pair)
    fmask = jnp.concatenate([jnp.array([True]), sp[1:] != sp[:-1]])
    new_src = jnp.where(fmask, sp // n_new, 0).astype(jnp.int32)
    new_dst = jnp.where(fmask, sp % n_new, 0).astype(jnp.int32)
    x1 = jax.ops.segment_sum(x, cluster_j, num_segments=n) * score[:, None]
    new_batch = jnp.full((n,), 1, dtype=batch.dtype).at[cluster_j].set(batch)
    h = jax.nn.relu(_bn(_cheb(x1, new_src, new_dst, c1_W0, c1_W1, c1_b, n),
                        bn1_g, bn1_b))
    h = jax.nn.relu(_bn(_cheb(h, new_src, new_dst, c2_W0, c2_W1, c2_b, n),
                        bn2_g, bn2_b))
    gmp = jax.ops.segment_max(h, new_batch, num_segments=1)
    cnt = jax.ops.segment_sum(jnp.ones((h.shape[0], 1), dtype=h.dtype),
                              new_batch, num_segments=1)
    gap = jax.ops.segment_sum(h, new_batch, num_segments=1) / jnp.maximum(cnt, 1.0)
    g = jnp.concatenate([gmp, gap], axis=1)
    h = jax.nn.relu(_bn(g @ l1_W + l1_b, bn3_g, bn3_b))
    h = jax.nn.relu(_bn(h @ l2_W + l2_b, bn4_g, bn4_b))
    feature = h
    out = jax.nn.log_softmax(jax.nn.relu(h @ l3_W + l3_b), axis=-1)
    return (out, feature)


# speedup vs baseline: 2.5880x; 2.5880x over previous
import functools

import jax
import jax.numpy as jnp
import numpy as np
from jax.experimental import pallas as pl

_D_IN = 1025
_EPS_BN = 1e-5


def _mm_body(a_ref, b_ref, o_ref):
    o_ref[...] = jnp.dot(a_ref[...], b_ref[...],
                         preferred_element_type=jnp.float32)


def _pallas_mm(a, b, block_m=400):
    m, k = a.shape
    k2, n = b.shape
    assert k == k2 and m % block_m == 0
    grid = (m // block_m,)
    return pl.pallas_call(
        _mm_body,
        grid=grid,
        in_specs=[
            pl.BlockSpec((block_m, k), lambda i: (i, 0)),
            pl.BlockSpec((k, n), lambda i: (0, 0)),
        ],
        out_specs=pl.BlockSpec((block_m, n), lambda i: (i, 0)),
        out_shape=jax.ShapeDtypeStruct((m, n), jnp.float32),
    )(a, b)


def _seg_softmax(e, idx, n):
    m = jax.ops.segment_max(e, idx, num_segments=n)
    m = jnp.where(jnp.isfinite(m), m, 0.0)
    ex = jnp.exp(e - m[idx])
    s = jax.ops.segment_sum(ex, idx, num_segments=n)
    return ex / (s[idx] + 1e-16)


def _greedy_merge(e, src, dst, n):
    E = e.shape[0]
    perm = jnp.argsort(-e)
    rank = jnp.zeros((E,), jnp.int32).at[perm].set(
        jnp.arange(E, dtype=jnp.int32))
    BIG = jnp.int32(E)

    def cond(state):
        alive, _ = state
        return jnp.any(alive)

    def body(state):
        alive, matched_e = state
        r = jnp.where(alive, rank, BIG)
        best = jnp.full((n,), BIG, jnp.int32)
        best = best.at[src].min(r)
        best = best.at[dst].min(r)
        win = alive & (rank == best[src]) & (rank == best[dst])
        dead = jnp.zeros((n,), bool).at[src].max(win).at[dst].max(win)
        alive = alive & ~dead[src] & ~dead[dst]
        return alive, matched_e | win

    alive, matched_e = jax.lax.while_loop(
        cond, body, (jnp.ones((E,), bool), jnp.zeros((E,), bool)))

    m_in_perm = matched_e[perm]
    ii = jnp.cumsum(m_in_perm.astype(jnp.int32)) - 1
    n_matched = ii[-1] + 1
    dump = jnp.int32(n)
    tgt_s = jnp.where(m_in_perm, src[perm], dump)
    tgt_t = jnp.where(m_in_perm, dst[perm], dump)
    cluster = jnp.zeros((n + 1,), jnp.int32).at[tgt_s].set(ii).at[tgt_t].set(ii)
    node_matched = (jnp.zeros((n + 1,), bool)
                    .at[tgt_s].set(True).at[tgt_t].set(True))
    cluster = cluster[:n]
    node_matched = node_matched[:n]
    tgt_i = jnp.where(m_in_perm, ii, dump)
    score = jnp.zeros((n + 1,), e.dtype).at[tgt_i].set(e[perm])[:n]
    score = jnp.where(jnp.arange(n) < n_matched, score, 1.0)
    unmatched = ~node_matched
    csum = jnp.cumsum(unmatched.astype(jnp.int32))
    cluster = jnp.where(unmatched, n_matched + csum - 1, cluster)
    n_new = n_matched + csum[-1]
    return cluster, score, n_matched, n_new


def _cheb(x, src, dst, W0, W1, bias, n):
    w = jnp.where(src == dst, 0.0, 1.0).astype(x.dtype)
    deg = jax.ops.segment_sum(w, src, num_segments=n)
    dis = jnp.where(deg > 0, 1.0 / jnp.sqrt(jnp.maximum(deg, 1e-12)), 0.0)
    norm = -dis[src] * w * dis[dst]
    xw1 = _pallas_mm(x, W1)
    tx1 = jax.ops.segment_sum(norm[:, None] * xw1[src], dst, num_segments=n)
    return _pallas_mm(x, W0) + tx1 + bias


def _bn(h, g, b):
    return h * (g / np.sqrt(1.0 + _EPS_BN)) + b


def kernel(x, edge_index, batch, A, pool_W, pool_b, c1_W0, c1_W1, c1_b,
           bn1_g, bn1_b, c2_W0, c2_W1, c2_b, bn2_g, bn2_b, l1_W, l1_b,
           bn3_g, bn3_b, l2_W, l2_b, bn4_g, bn4_b, l3_W, l3_b):
    src, dst = edge_index[0], edge_index[1]
    n = x.shape[0]
    ab = _pallas_mm(x, jnp.concatenate(
        [pool_W[:_D_IN, :], pool_W[_D_IN:, :]], axis=1))
    raw = ab[src, 0] + ab[dst, 1] + pool_b[0]
    e = _seg_softmax(raw, dst, n) + 0.5
    cluster_j, score, n_merged, n_new = _greedy_merge(e, src, dst, n)
    pair = cluster_j[src] * n_new + cluster_j[dst]
    sp = jnp.sort(pair)
    fmask = jnp.concatenate([jnp.array([True]), sp[1:] != sp[:-1]])
    new_src = jnp.where(fmask, sp // n_new, 0).astype(jnp.int32)
    new_dst = jnp.where(fmask, sp % n_new, 0).astype(jnp.int32)
    x1 = jax.ops.segment_sum(x, cluster_j, num_segments=n) * score[:, None]
    new_batch = jnp.full((n,), 1, dtype=batch.dtype).at[cluster_j].set(batch)
    h = jax.nn.relu(_bn(_cheb(x1, new_src, new_dst, c1_W0, c1_W1, c1_b, n),
                        bn1_g, bn1_b))
    h = jax.nn.relu(_bn(_cheb(h, new_src, new_dst, c2_W0, c2_W1, c2_b, n),
                        bn2_g, bn2_b))
    gmp = jax.ops.segment_max(h, new_batch, num_segments=1)
    cnt = jax.ops.segment_sum(jnp.ones((h.shape[0], 1), dtype=h.dtype),
                              new_batch, num_segments=1)
    gap = jax.ops.segment_sum(h, new_batch, num_segments=1) / jnp.maximum(cnt, 1.0)
    g = jnp.concatenate([gmp, gap], axis=1)
    h = jax.nn.relu(_bn(g @ l1_W + l1_b, bn3_g, bn3_b))
    h = jax.nn.relu(_bn(h @ l2_W + l2_b, bn4_g, bn4_b))
    feature = h
    out = jax.nn.log_softmax(jax.nn.relu(h @ l3_W + l3_b), axis=-1)
    return (out, feature)


# speedup vs baseline: 7.0315x; 2.7170x over previous
import functools

import jax
import jax.numpy as jnp
import numpy as np
from jax.experimental import pallas as pl

_D_IN = 1025
_EPS_BN = 1e-5


def _mm_body(a_ref, b_ref, o_ref):
    o_ref[...] = jnp.dot(a_ref[...], b_ref[...],
                         preferred_element_type=jnp.float32)


def _pallas_mm(a, b, block_m=400):
    m, k = a.shape
    k2, n = b.shape
    assert k == k2 and m % block_m == 0
    grid = (m // block_m,)
    return pl.pallas_call(
        _mm_body,
        grid=grid,
        in_specs=[
            pl.BlockSpec((block_m, k), lambda i: (i, 0)),
            pl.BlockSpec((k, n), lambda i: (0, 0)),
        ],
        out_specs=pl.BlockSpec((block_m, n), lambda i: (i, 0)),
        out_shape=jax.ShapeDtypeStruct((m, n), jnp.float32),
    )(a, b)


def _seg_softmax(e, idx, n):
    m = jax.ops.segment_max(e, idx, num_segments=n)
    m = jnp.where(jnp.isfinite(m), m, 0.0)
    ex = jnp.exp(e - m[idx])
    s = jax.ops.segment_sum(ex, idx, num_segments=n)
    return ex / (s[idx] + 1e-16)


def _greedy_merge(e, src, dst, n):
    E = e.shape[0]
    perm = jnp.argsort(-e)
    rank = jnp.zeros((E,), jnp.int32).at[perm].max(
        jnp.arange(E, dtype=jnp.int32))
    BIG = jnp.int32(E)
    one = jnp.int32(1)

    def cond(state):
        alive_i, _ = state
        return jnp.any(alive_i == 1)

    def body(state):
        alive_i, matched_i = state
        r = jnp.where(alive_i == 1, rank, BIG)
        best = jnp.full((n,), BIG, jnp.int32)
        best = best.at[src].min(r)
        best = best.at[dst].min(r)
        win_i = (jnp.logical_and(rank == best[src], rank == best[dst])
                 .astype(jnp.int32) * alive_i)
        dead = jnp.zeros((n,), jnp.int32).at[src].max(win_i).at[dst].max(win_i)
        alive_i = alive_i * (one - dead[src]) * (one - dead[dst])
        return alive_i, jnp.maximum(matched_i, win_i)

    _, matched_i = jax.lax.while_loop(
        cond, body,
        (jnp.ones((E,), jnp.int32), jnp.zeros((E,), jnp.int32)))

    m_in_perm = matched_i[perm]
    ii = jnp.cumsum(m_in_perm) - 1
    n_matched = ii[-1] + 1
    dump = jnp.int32(n)
    mp = m_in_perm == 1
    tgt_s = jnp.where(mp, src[perm], dump)
    tgt_t = jnp.where(mp, dst[perm], dump)
    cluster = jnp.zeros((n + 1,), jnp.int32).at[tgt_s].max(ii).at[tgt_t].max(ii)
    node_matched = (jnp.zeros((n + 1,), jnp.int32)
                    .at[tgt_s].max(one).at[tgt_t].max(one))
    cluster = cluster[:n]
    node_matched = node_matched[:n]
    tgt_i = jnp.where(mp, ii, dump)
    score = jnp.zeros((n + 1,), e.dtype).at[tgt_i].max(e[perm])[:n]
    score = jnp.where(jnp.arange(n) < n_matched, score, 1.0)
    unmatched = one - node_matched
    csum = jnp.cumsum(unmatched)
    cluster = jnp.where(unmatched == 1, n_matched + csum - 1, cluster)
    n_new = n_matched + csum[-1]
    return cluster, score, n_matched, n_new


def _cheb(x, src, dst, W0, W1, bias, n):
    w = jnp.where(src == dst, 0.0, 1.0).astype(x.dtype)
    deg = jax.ops.segment_sum(w, src, num_segments=n)
    dis = jnp.where(deg > 0, 1.0 / jnp.sqrt(jnp.maximum(deg, 1e-12)), 0.0)
    norm = -dis[src] * w * dis[dst]
    xw1 = _pallas_mm(x, W1)
    tx1 = jax.ops.segment_sum(norm[:, None] * xw1[src], dst, num_segments=n)
    return _pallas_mm(x, W0) + tx1 + bias


def _bn(h, g, b):
    return h * (g / np.sqrt(1.0 + _EPS_BN)) + b


def kernel(x, edge_index, batch, A, pool_W, pool_b, c1_W0, c1_W1, c1_b,
           bn1_g, bn1_b, c2_W0, c2_W1, c2_b, bn2_g, bn2_b, l1_W, l1_b,
           bn3_g, bn3_b, l2_W, l2_b, bn4_g, bn4_b, l3_W, l3_b):
    src, dst = edge_index[0], edge_index[1]
    n = x.shape[0]
    ab = _pallas_mm(x, jnp.concatenate(
        [pool_W[:_D_IN, :], pool_W[_D_IN:, :]], axis=1))
    raw = ab[src, 0] + ab[dst, 1] + pool_b[0]
    e = _seg_softmax(raw, dst, n) + 0.5
    cluster_j, score, n_merged, n_new = _greedy_merge(e, src, dst, n)
    pair = cluster_j[src] * n_new + cluster_j[dst]
    sp = jnp.sort(pair)
    fmask = jnp.concatenate([jnp.array([True]), sp[1:] != sp[:-1]])
    new_src = jnp.where(fmask, sp // n_new, 0).astype(jnp.int32)
    new_dst = jnp.where(fmask, sp % n_new, 0).astype(jnp.int32)
    x1 = jax.ops.segment_sum(x, cluster_j, num_segments=n) * score[:, None]
    new_batch = jnp.where(jnp.arange(n) < n_new, batch[0],
                          jnp.ones((), batch.dtype))
    h = jax.nn.relu(_bn(_cheb(x1, new_src, new_dst, c1_W0, c1_W1, c1_b, n),
                        bn1_g, bn1_b))
    h = jax.nn.relu(_bn(_cheb(h, new_src, new_dst, c2_W0, c2_W1, c2_b, n),
                        bn2_g, bn2_b))
    gmp = jax.ops.segment_max(h, new_batch, num_segments=1)
    cnt = jax.ops.segment_sum(jnp.ones((h.shape[0], 1), dtype=h.dtype),
                              new_batch, num_segments=1)
    gap = jax.ops.segment_sum(h, new_batch, num_segments=1) / jnp.maximum(cnt, 1.0)
    g = jnp.concatenate([gmp, gap], axis=1)
    h = jax.nn.relu(_bn(g @ l1_W + l1_b, bn3_g, bn3_b))
    h = jax.nn.relu(_bn(h @ l2_W + l2_b, bn4_g, bn4_b))
    feature = h
    out = jax.nn.log_softmax(jax.nn.relu(h @ l3_W + l3_b), axis=-1)
    return (out, feature)


# speedup vs baseline: 7.2844x; 1.0360x over previous
import functools

import jax
import jax.numpy as jnp
import numpy as np
from jax import lax
from jax.experimental import pallas as pl
from jax.experimental.pallas import tpu as pltpu
from jax.experimental.pallas import tpu_sc as plsc

_D_IN = 1025
_EPS_BN = 1e-5

_NC = 2
_NS = 16
_N = 10000
_E = 160000
_DC = 128
_NCHUNK = 4
_B = 128
_EPAD = 163840
_NB = _EPAD // (_NC * _NS * _B)
_ROWS = 10240
_STRIPE = _ROWS // _NS
_ZR = 64


def _sc_agg_body(tab, srcc, dstc, out, idx_v, dsti_v, rows_v, zero_v, accum):
    cid = lax.axis_index("c")
    sid = lax.axis_index("s")
    zvec = jnp.zeros((16,), jnp.float32)

    def _zrow(i, carry):
        for j in range(_DC // 16):
            zero_v[i, pl.ds(j * 16, 16)] = zvec
        return carry

    lax.fori_loop(0, _ZR, _zrow, 0)
    pltpu.sync_copy(dstc.at[cid, sid], dsti_v)

    for c in range(_NCHUNK):
        for z in range(_STRIPE // _ZR):
            pltpu.sync_copy(zero_v,
                            accum.at[pl.ds(sid * _STRIPE + z * _ZR, _ZR)])
        pltpu.sync_copy(srcc.at[cid, sid, c], idx_v)
        plsc.subcore_barrier()

        def _batch(b, carry):
            pltpu.sync_copy(tab.at[idx_v.at[b]], rows_v)
            pltpu.sync_copy(rows_v, accum.at[dsti_v.at[b]], add=True)
            return carry

        lax.fori_loop(0, _NB, _batch, 0)
        plsc.subcore_barrier()
        pltpu.sync_copy(accum.at[pl.ds(sid * _STRIPE, _STRIPE)],
                        out.at[cid, c, pl.ds(sid * _STRIPE, _STRIPE)])
        plsc.subcore_barrier()


def _sc_edge_agg(tab4, srcc, dstc):
    mesh = plsc.VectorSubcoreMesh(core_axis_name="c", subcore_axis_name="s")
    return pl.kernel(
        _sc_agg_body,
        out_type=jax.ShapeDtypeStruct((_NC, _NCHUNK, _ROWS, _DC),
                                      jnp.float32),
        mesh=mesh,
        scratch_types=[
            pltpu.VMEM((_NB, _B), jnp.int32),
            pltpu.VMEM((_NB, _B), jnp.int32),
            pltpu.VMEM((_B, _DC), jnp.float32),
            pltpu.VMEM((_ZR, _DC), jnp.float32),
            pltpu.VMEM_SHARED((_ROWS, _DC), jnp.float32),
        ],
    )(tab4, srcc, dstc)


def _edge_aggregate(xw1, new_src, new_dst, dis):
    y = dis[:, None] * xw1
    tab4 = (y.reshape(_N, _NCHUNK, _DC)
             .transpose(1, 0, 2)
             .reshape(_NCHUNK * _N, _DC))
    npad = _EPAD - _E
    src_p = jnp.concatenate(
        [new_src, jnp.zeros((npad,), jnp.int32)])
    effd = jnp.where(new_src == new_dst, jnp.int32(_N), new_dst)
    dst_p = jnp.concatenate(
        [effd, jnp.full((npad,), jnp.int32(_N))])
    offs = (jnp.arange(_NCHUNK, dtype=jnp.int32) * _N).reshape(1, 1, -1, 1, 1)
    srcc = src_p.reshape(_NC, _NS, 1, _NB, _B) + offs
    dstc = dst_p.reshape(_NC, _NS, _NB, _B)
    part = _sc_edge_agg(tab4, srcc, dstc)
    s = part[0] + part[1]
    agg = s[:, :_N, :].transpose(1, 0, 2).reshape(_N, _NCHUNK * _DC)
    return -dis[:, None] * agg


def _mm_body(a_ref, b_ref, o_ref):
    o_ref[...] = jnp.dot(a_ref[...], b_ref[...],
                         preferred_element_type=jnp.float32)


def _pallas_mm(a, b, block_m=400):
    m, k = a.shape
    k2, n = b.shape
    assert k == k2 and m % block_m == 0
    grid = (m // block_m,)
    return pl.pallas_call(
        _mm_body,
        grid=grid,
        in_specs=[
            pl.BlockSpec((block_m, k), lambda i: (i, 0)),
            pl.BlockSpec((k, n), lambda i: (0, 0)),
        ],
        out_specs=pl.BlockSpec((block_m, n), lambda i: (i, 0)),
        out_shape=jax.ShapeDtypeStruct((m, n), jnp.float32),
    )(a, b)


def _seg_softmax(e, idx, n):
    m = jax.ops.segment_max(e, idx, num_segments=n)
    m = jnp.where(jnp.isfinite(m), m, 0.0)
    ex = jnp.exp(e - m[idx])
    s = jax.ops.segment_sum(ex, idx, num_segments=n)
    return ex / (s[idx] + 1e-16)


def _greedy_merge(e, src, dst, n):
    E = e.shape[0]
    perm = jnp.argsort(-e)
    rank = jnp.zeros((E,), jnp.int32).at[perm].max(
        jnp.arange(E, dtype=jnp.int32))
    BIG = jnp.int32(E)
    one = jnp.int32(1)

    def cond(state):
        alive_i, _ = state
        return jnp.any(alive_i == 1)

    def body(state):
        alive_i, matched_i = state
        r = jnp.where(alive_i == 1, rank, BIG)
        best = jnp.full((n,), BIG, jnp.int32)
        best = best.at[src].min(r)
        best = best.at[dst].min(r)
        win_i = (jnp.logical_and(rank == best[src], rank == best[dst])
                 .astype(jnp.int32) * alive_i)
        dead = jnp.zeros((n,), jnp.int32).at[src].max(win_i).at[dst].max(win_i)
        alive_i = alive_i * (one - dead[src]) * (one - dead[dst])
        return alive_i, jnp.maximum(matched_i, win_i)

    _, matched_i = jax.lax.while_loop(
        cond, body,
        (jnp.ones((E,), jnp.int32), jnp.zeros((E,), jnp.int32)))

    m_in_perm = matched_i[perm]
    ii = jnp.cumsum(m_in_perm) - 1
    n_matched = ii[-1] + 1
    dump = jnp.int32(n)
    mp = m_in_perm == 1
    tgt_s = jnp.where(mp, src[perm], dump)
    tgt_t = jnp.where(mp, dst[perm], dump)
    cluster = jnp.zeros((n + 1,), jnp.int32).at[tgt_s].max(ii).at[tgt_t].max(ii)
    node_matched = (jnp.zeros((n + 1,), jnp.int32)
                    .at[tgt_s].max(one).at[tgt_t].max(one))
    cluster = cluster[:n]
    node_matched = node_matched[:n]
    tgt_i = jnp.where(mp, ii, dump)
    score = jnp.zeros((n + 1,), e.dtype).at[tgt_i].max(e[perm])[:n]
    score = jnp.where(jnp.arange(n) < n_matched, score, 1.0)
    unmatched = one - node_matched
    csum = jnp.cumsum(unmatched)
    cluster = jnp.where(unmatched == 1, n_matched + csum - 1, cluster)
    n_new = n_matched + csum[-1]
    return cluster, score, n_matched, n_new


def _cheb(x, src, dst, W0, W1, bias, n):
    w = jnp.where(src == dst, 0.0, 1.0).astype(x.dtype)
    deg = jax.ops.segment_sum(w, src, num_segments=n)
    dis = jnp.where(deg > 0, 1.0 / jnp.sqrt(jnp.maximum(deg, 1e-12)), 0.0)
    xw1 = _pallas_mm(x, W1)
    tx1 = _edge_aggregate(xw1, src, dst, dis)
    return _pallas_mm(x, W0) + tx1 + bias


def _bn(h, g, b):
    return h * (g / np.sqrt(1.0 + _EPS_BN)) + b


def kernel(x, edge_index, batch, A, pool_W, pool_b, c1_W0, c1_W1, c1_b,
           bn1_g, bn1_b, c2_W0, c2_W1, c2_b, bn2_g, bn2_b, l1_W, l1_b,
           bn3_g, bn3_b, l2_W, l2_b, bn4_g, bn4_b, l3_W, l3_b):
    src, dst = edge_index[0], edge_index[1]
    n = x.shape[0]
    ab = _pallas_mm(x, jnp.concatenate(
        [pool_W[:_D_IN, :], pool_W[_D_IN:, :]], axis=1))
    raw = ab[src, 0] + ab[dst, 1] + pool_b[0]
    e = _seg_softmax(raw, dst, n) + 0.5
    cluster_j, score, n_merged, n_new = _greedy_merge(e, src, dst, n)
    pair = cluster_j[src] * n_new + cluster_j[dst]
    sp = jnp.sort(pair)
    fmask = jnp.concatenate([jnp.array([True]), sp[1:] != sp[:-1]])
    new_src = jnp.where(fmask, sp // n_new, 0).astype(jnp.int32)
    new_dst = jnp.where(fmask, sp % n_new, 0).astype(jnp.int32)
    x1 = jax.ops.segment_sum(x, cluster_j, num_segments=n) * score[:, None]
    new_batch = jnp.where(jnp.arange(n) < n_new, batch[0],
                          jnp.ones((), batch.dtype))
    h = jax.nn.relu(_bn(_cheb(x1, new_src, new_dst, c1_W0, c1_W1, c1_b, n),
                        bn1_g, bn1_b))
    h = jax.nn.relu(_bn(_cheb(h, new_src, new_dst, c2_W0, c2_W1, c2_b, n),
                        bn2_g, bn2_b))
    gmp = jax.ops.segment_max(h, new_batch, num_segments=1)
    cnt = jax.ops.segment_sum(jnp.ones((h.shape[0], 1), dtype=h.dtype),
                              new_batch, num_segments=1)
    gap = jax.ops.segment_sum(h, new_batch, num_segments=1) / jnp.maximum(cnt, 1.0)
    g = jnp.concatenate([gmp, gap], axis=1)
    h = jax.nn.relu(_bn(g @ l1_W + l1_b, bn3_g, bn3_b))
    h = jax.nn.relu(_bn(h @ l2_W + l2_b, bn4_g, bn4_b))
    feature = h
    out = jax.nn.log_softmax(jax.nn.relu(h @ l3_W + l3_b), axis=-1)
    return (out, feature)


# speedup vs baseline: 7.9564x; 1.0922x over previous
import functools

import jax
import jax.numpy as jnp
import numpy as np
from jax import lax
from jax.experimental import pallas as pl
from jax.experimental.pallas import tpu as pltpu
from jax.experimental.pallas import tpu_sc as plsc

_D_IN = 1025
_EPS_BN = 1e-5

_NC = 2
_NS = 16
_N = 10000
_E = 160000
_DC = 128
_NCHUNK = 4
_B = 128
_EPAD = 163840
_NB = _EPAD // (_NC * _NS * _B)
_ROWS = 10240
_STRIPE = _ROWS // _NS
_ZR = 64


def _sc_agg_body(tab, srcc, dstc, out, idx_v, dsti_v, rows_v, zero_v, accum):
    cid = lax.axis_index("c")
    sid = lax.axis_index("s")
    zvec = jnp.zeros((16,), jnp.float32)

    def _zrow(i, carry):
        for j in range(_DC // 16):
            zero_v[i, pl.ds(j * 16, 16)] = zvec
        return carry

    lax.fori_loop(0, _ZR, _zrow, 0)
    pltpu.sync_copy(dstc.at[cid, sid], dsti_v)

    for c in range(_NCHUNK):
        for z in range(_STRIPE // _ZR):
            pltpu.sync_copy(zero_v,
                            accum.at[pl.ds(sid * _STRIPE + z * _ZR, _ZR)])
        pltpu.sync_copy(srcc.at[cid, sid, c], idx_v)
        plsc.subcore_barrier()

        def _batch(b, carry):
            pltpu.sync_copy(tab.at[idx_v.at[b]], rows_v)
            pltpu.sync_copy(rows_v, accum.at[dsti_v.at[b]], add=True)
            return carry

        lax.fori_loop(0, _NB, _batch, 0)
        plsc.subcore_barrier()
        pltpu.sync_copy(accum.at[pl.ds(sid * _STRIPE, _STRIPE)],
                        out.at[cid, c, pl.ds(sid * _STRIPE, _STRIPE)])
        plsc.subcore_barrier()


def _sc_edge_agg(tab4, srcc, dstc):
    mesh = plsc.VectorSubcoreMesh(core_axis_name="c", subcore_axis_name="s")
    return pl.kernel(
        _sc_agg_body,
        out_type=jax.ShapeDtypeStruct((_NC, _NCHUNK, _ROWS, _DC),
                                      jnp.float32),
        mesh=mesh,
        scratch_types=[
            pltpu.VMEM((_NB, _B), jnp.int32),
            pltpu.VMEM((_NB, _B), jnp.int32),
            pltpu.VMEM((_B, _DC), jnp.float32),
            pltpu.VMEM((_ZR, _DC), jnp.float32),
            pltpu.VMEM_SHARED((_ROWS, _DC), jnp.float32),
        ],
    )(tab4, srcc, dstc)


def _edge_aggregate(xw1, new_src, new_dst, dis):
    y = dis[:, None] * xw1
    tab4 = (y.reshape(_N, _NCHUNK, _DC)
             .transpose(1, 0, 2)
             .reshape(_NCHUNK * _N, _DC))
    npad = _EPAD - _E
    src_p = jnp.concatenate(
        [new_src, jnp.zeros((npad,), jnp.int32)])
    effd = jnp.where(new_src == new_dst, jnp.int32(_N), new_dst)
    dst_p = jnp.concatenate(
        [effd, jnp.full((npad,), jnp.int32(_N))])
    offs = (jnp.arange(_NCHUNK, dtype=jnp.int32) * _N).reshape(1, 1, -1, 1, 1)
    srcc = src_p.reshape(_NC, _NS, 1, _NB, _B) + offs
    dstc = dst_p.reshape(_NC, _NS, _NB, _B)
    part = _sc_edge_agg(tab4, srcc, dstc)
    s = part[0] + part[1]
    agg = s[:, :_N, :].transpose(1, 0, 2).reshape(_N, _NCHUNK * _DC)
    return -dis[:, None] * agg


def _mm_body(a_ref, b_ref, o_ref):
    o_ref[...] = jnp.dot(a_ref[...], b_ref[...],
                         preferred_element_type=jnp.float32)


def _pallas_mm(a, b, block_m=400):
    m, k = a.shape
    k2, n = b.shape
    assert k == k2 and m % block_m == 0
    grid = (m // block_m,)
    return pl.pallas_call(
        _mm_body,
        grid=grid,
        in_specs=[
            pl.BlockSpec((block_m, k), lambda i: (i, 0)),
            pl.BlockSpec((k, n), lambda i: (0, 0)),
        ],
        out_specs=pl.BlockSpec((block_m, n), lambda i: (i, 0)),
        out_shape=jax.ShapeDtypeStruct((m, n), jnp.float32),
    )(a, b)


def _seg_softmax(e, idx, n):
    m = jax.ops.segment_max(e, idx, num_segments=n)
    m = jnp.where(jnp.isfinite(m), m, 0.0)
    ex = jnp.exp(e - m[idx])
    s = jax.ops.segment_sum(ex, idx, num_segments=n)
    return ex / (s[idx] + 1e-16)


def _greedy_merge(e, src, dst, n):
    E = e.shape[0]
    perm = jnp.argsort(-e)
    rank = jnp.zeros((E,), jnp.int32).at[perm].max(
        jnp.arange(E, dtype=jnp.int32))
    BIG = jnp.int32(E)
    one = jnp.int32(1)
    idx2 = jnp.concatenate([src, dst])

    def cond(state):
        alive_i, _ = state
        return jnp.any(alive_i == 1)

    def body(state):
        alive_i, matched_i = state
        r = jnp.where(alive_i == 1, rank, BIG)
        r2 = jnp.concatenate([r, r])
        best = jnp.full((n,), BIG, jnp.int32).at[idx2].min(r2)
        b2 = best[idx2]
        win_i = (jnp.logical_and(rank == b2[:E], rank == b2[E:])
                 .astype(jnp.int32) * alive_i)
        w2 = jnp.concatenate([win_i, win_i])
        dead = jnp.zeros((n,), jnp.int32).at[idx2].max(w2)
        d2 = dead[idx2]
        alive_i = alive_i * (one - d2[:E]) * (one - d2[E:])
        return alive_i, jnp.maximum(matched_i, win_i)

    _, matched_i = jax.lax.while_loop(
        cond, body,
        (jnp.ones((E,), jnp.int32), jnp.zeros((E,), jnp.int32)))

    m_in_perm = matched_i[perm]
    ii = jnp.cumsum(m_in_perm) - 1
    n_matched = ii[-1] + 1
    dump = jnp.int32(n)
    mp = m_in_perm == 1
    tgt_s = jnp.where(mp, src[perm], dump)
    tgt_t = jnp.where(mp, dst[perm], dump)
    cluster = jnp.zeros((n + 1,), jnp.int32).at[tgt_s].max(ii).at[tgt_t].max(ii)
    node_matched = (jnp.zeros((n + 1,), jnp.int32)
                    .at[tgt_s].max(one).at[tgt_t].max(one))
    cluster = cluster[:n]
    node_matched = node_matched[:n]
    tgt_i = jnp.where(mp, ii, dump)
    score = jnp.zeros((n + 1,), e.dtype).at[tgt_i].max(e[perm])[:n]
    score = jnp.where(jnp.arange(n) < n_matched, score, 1.0)
    unmatched = one - node_matched
    csum = jnp.cumsum(unmatched)
    cluster = jnp.where(unmatched == 1, n_matched + csum - 1, cluster)
    n_new = n_matched + csum[-1]
    return cluster, score, n_matched, n_new


def _cheb(x, src, dst, W0, W1, bias, n):
    w = jnp.where(src == dst, 0.0, 1.0).astype(x.dtype)
    deg = jax.ops.segment_sum(w, src, num_segments=n)
    dis = jnp.where(deg > 0, 1.0 / jnp.sqrt(jnp.maximum(deg, 1e-12)), 0.0)
    xw1 = _pallas_mm(x, W1)
    tx1 = _edge_aggregate(xw1, src, dst, dis)
    return _pallas_mm(x, W0) + tx1 + bias


def _bn(h, g, b):
    return h * (g / np.sqrt(1.0 + _EPS_BN)) + b


def kernel(x, edge_index, batch, A, pool_W, pool_b, c1_W0, c1_W1, c1_b,
           bn1_g, bn1_b, c2_W0, c2_W1, c2_b, bn2_g, bn2_b, l1_W, l1_b,
           bn3_g, bn3_b, l2_W, l2_b, bn4_g, bn4_b, l3_W, l3_b):
    src, dst = edge_index[0], edge_index[1]
    n = x.shape[0]
    ab = _pallas_mm(x, jnp.concatenate(
        [pool_W[:_D_IN, :], pool_W[_D_IN:, :]], axis=1))
    raw = ab[src, 0] + ab[dst, 1] + pool_b[0]
    e = _seg_softmax(raw, dst, n) + 0.5
    cluster_j, score, n_merged, n_new = _greedy_merge(e, src, dst, n)
    pair = cluster_j[src] * n_new + cluster_j[dst]
    sp = jnp.sort(pair)
    fmask = jnp.concatenate([jnp.array([True]), sp[1:] != sp[:-1]])
    new_src = jnp.where(fmask, sp // n_new, 0).astype(jnp.int32)
    new_dst = jnp.where(fmask, sp % n_new, 0).astype(jnp.int32)
    x1 = jax.ops.segment_sum(x, cluster_j, num_segments=n) * score[:, None]
    new_batch = jnp.where(jnp.arange(n) < n_new, batch[0],
                          jnp.ones((), batch.dtype))
    h = jax.nn.relu(_bn(_cheb(x1, new_src, new_dst, c1_W0, c1_W1, c1_b, n),
                        bn1_g, bn1_b))
    h = jax.nn.relu(_bn(_cheb(h, new_src, new_dst, c2_W0, c2_W1, c2_b, n),
                        bn2_g, bn2_b))
    gmp = jax.ops.segment_max(h, new_batch, num_segments=1)
    cnt = jax.ops.segment_sum(jnp.ones((h.shape[0], 1), dtype=h.dtype),
                              new_batch, num_segments=1)
    gap = jax.ops.segment_sum(h, new_batch, num_segments=1) / jnp.maximum(cnt, 1.0)
    g = jnp.concatenate([gmp, gap], axis=1)
    h = jax.nn.relu(_bn(g @ l1_W + l1_b, bn3_g, bn3_b))
    h = jax.nn.relu(_bn(h @ l2_W + l2_b, bn4_g, bn4_b))
    feature = h
    out = jax.nn.log_softmax(jax.nn.relu(h @ l3_W + l3_b), axis=-1)
    return (out, feature)


# speedup vs baseline: 27.4613x; 3.4515x over previous
import functools

import jax
import jax.numpy as jnp
import numpy as np
from jax import lax
from jax.experimental import pallas as pl
from jax.experimental.pallas import tpu as pltpu
from jax.experimental.pallas import tpu_sc as plsc

_D_IN = 1025
_EPS_BN = 1e-5

from jax.experimental.compute_on import compute_on


@functools.partial(jax.jit, static_argnames=())
@compute_on("tpu_sparsecore")
def _sc_take(a, idx):
    return jnp.take(a, idx, axis=0, mode="clip")

_NC = 2
_NS = 16
_N = 10000
_E = 160000
_DC = 128
_NCHUNK = 4
_B = 128
_EPAD = 163840
_NB = _EPAD // (_NC * _NS * _B)
_ROWS = 10240
_STRIPE = _ROWS // _NS
_ZR = 64


def _sc_agg_body(tab, srcc, dstc, out, idx_v, dsti_v, rows_v, zero_v, accum):
    cid = lax.axis_index("c")
    sid = lax.axis_index("s")
    zvec = jnp.zeros((16,), jnp.float32)

    def _zrow(i, carry):
        for j in range(_DC // 16):
            zero_v[i, pl.ds(j * 16, 16)] = zvec
        return carry

    lax.fori_loop(0, _ZR, _zrow, 0)
    pltpu.sync_copy(dstc.at[cid, sid], dsti_v)

    for c in range(_NCHUNK):
        for z in range(_STRIPE // _ZR):
            pltpu.sync_copy(zero_v,
                            accum.at[pl.ds(sid * _STRIPE + z * _ZR, _ZR)])
        pltpu.sync_copy(srcc.at[cid, sid, c], idx_v)
        plsc.subcore_barrier()

        def _batch(b, carry):
            pltpu.sync_copy(tab.at[idx_v.at[b]], rows_v)
            pltpu.sync_copy(rows_v, accum.at[dsti_v.at[b]], add=True)
            return carry

        lax.fori_loop(0, _NB, _batch, 0)
        plsc.subcore_barrier()
        pltpu.sync_copy(accum.at[pl.ds(sid * _STRIPE, _STRIPE)],
                        out.at[cid, c, pl.ds(sid * _STRIPE, _STRIPE)])
        plsc.subcore_barrier()


def _sc_edge_agg(tab4, srcc, dstc):
    mesh = plsc.VectorSubcoreMesh(core_axis_name="c", subcore_axis_name="s")
    return pl.kernel(
        _sc_agg_body,
        out_type=jax.ShapeDtypeStruct((_NC, _NCHUNK, _ROWS, _DC),
                                      jnp.float32),
        mesh=mesh,
        scratch_types=[
            pltpu.VMEM((_NB, _B), jnp.int32),
            pltpu.VMEM((_NB, _B), jnp.int32),
            pltpu.VMEM((_B, _DC), jnp.float32),
            pltpu.VMEM((_ZR, _DC), jnp.float32),
            pltpu.VMEM_SHARED((_ROWS, _DC), jnp.float32),
        ],
    )(tab4, srcc, dstc)


def _edge_aggregate(xw1, new_src, new_dst, dis):
    y = dis[:, None] * xw1
    tab4 = (y.reshape(_N, _NCHUNK, _DC)
             .transpose(1, 0, 2)
             .reshape(_NCHUNK * _N, _DC))
    npad = _EPAD - _E
    src_p = jnp.concatenate(
        [new_src, jnp.zeros((npad,), jnp.int32)])
    effd = jnp.where(new_src == new_dst, jnp.int32(_N), new_dst)
    dst_p = jnp.concatenate(
        [effd, jnp.full((npad,), jnp.int32(_N))])
    offs = (jnp.arange(_NCHUNK, dtype=jnp.int32) * _N).reshape(1, 1, -1, 1, 1)
    srcc = src_p.reshape(_NC, _NS, 1, _NB, _B) + offs
    dstc = dst_p.reshape(_NC, _NS, _NB, _B)
    part = _sc_edge_agg(tab4, srcc, dstc)
    s = part[0] + part[1]
    agg = s[:, :_N, :].transpose(1, 0, 2).reshape(_N, _NCHUNK * _DC)
    return -dis[:, None] * agg


def _mm_body(a_ref, b_ref, o_ref):
    o_ref[...] = jnp.dot(a_ref[...], b_ref[...],
                         preferred_element_type=jnp.float32)


def _pallas_mm(a, b, block_m=400):
    m, k = a.shape
    k2, n = b.shape
    assert k == k2 and m % block_m == 0
    grid = (m // block_m,)
    return pl.pallas_call(
        _mm_body,
        grid=grid,
        in_specs=[
            pl.BlockSpec((block_m, k), lambda i: (i, 0)),
            pl.BlockSpec((k, n), lambda i: (0, 0)),
        ],
        out_specs=pl.BlockSpec((block_m, n), lambda i: (i, 0)),
        out_shape=jax.ShapeDtypeStruct((m, n), jnp.float32),
    )(a, b)


def _seg_softmax(e, idx, n):
    m = jax.ops.segment_max(e, idx, num_segments=n)
    m = jnp.where(jnp.isfinite(m), m, 0.0)
    ex = jnp.exp(e - _sc_take(m, idx))
    s = jax.ops.segment_sum(ex, idx, num_segments=n)
    return ex / (_sc_take(s, idx) + 1e-16)


def _greedy_merge(e, src, dst, n):
    E = e.shape[0]
    perm = jnp.argsort(-e)
    rank = jnp.zeros((E,), jnp.int32).at[perm].max(
        jnp.arange(E, dtype=jnp.int32))
    BIG = jnp.int32(E)
    one = jnp.int32(1)
    idx2 = jnp.concatenate([src, dst])

    def cond(state):
        alive_i, _ = state
        return jnp.any(alive_i == 1)

    def body(state):
        alive_i, matched_i = state
        r = jnp.where(alive_i == 1, rank, BIG)
        r2 = jnp.concatenate([r, r])
        best = jnp.full((n,), BIG, jnp.int32).at[idx2].min(r2)
        b2 = _sc_take(best, idx2)
        win_i = (jnp.logical_and(rank == b2[:E], rank == b2[E:])
                 .astype(jnp.int32) * alive_i)
        w2 = jnp.concatenate([win_i, win_i])
        dead = jnp.zeros((n,), jnp.int32).at[idx2].max(w2)
        d2 = _sc_take(dead, idx2)
        alive_i = alive_i * (one - d2[:E]) * (one - d2[E:])
        return alive_i, jnp.maximum(matched_i, win_i)

    _, matched_i = jax.lax.while_loop(
        cond, body,
        (jnp.ones((E,), jnp.int32), jnp.zeros((E,), jnp.int32)))

    m_in_perm = _sc_take(matched_i, perm)
    ii = jnp.cumsum(m_in_perm) - 1
    n_matched = ii[-1] + 1
    dump = jnp.int32(n)
    mp = m_in_perm == 1
    tgt_s = jnp.where(mp, _sc_take(src, perm), dump)
    tgt_t = jnp.where(mp, _sc_take(dst, perm), dump)
    cluster = jnp.zeros((n + 1,), jnp.int32).at[tgt_s].max(ii).at[tgt_t].max(ii)
    node_matched = (jnp.zeros((n + 1,), jnp.int32)
                    .at[tgt_s].max(one).at[tgt_t].max(one))
    cluster = cluster[:n]
    node_matched = node_matched[:n]
    tgt_i = jnp.where(mp, ii, dump)
    score = jnp.zeros((n + 1,), e.dtype).at[tgt_i].max(_sc_take(e, perm))[:n]
    score = jnp.where(jnp.arange(n) < n_matched, score, 1.0)
    unmatched = one - node_matched
    csum = jnp.cumsum(unmatched)
    cluster = jnp.where(unmatched == 1, n_matched + csum - 1, cluster)
    n_new = n_matched + csum[-1]
    return cluster, score, n_matched, n_new


def _cheb(x, src, dst, W0, W1, bias, n):
    w = jnp.where(src == dst, 0.0, 1.0).astype(x.dtype)
    deg = jax.ops.segment_sum(w, src, num_segments=n)
    dis = jnp.where(deg > 0, 1.0 / jnp.sqrt(jnp.maximum(deg, 1e-12)), 0.0)
    xw1 = _pallas_mm(x, W1)
    tx1 = _edge_aggregate(xw1, src, dst, dis)
    return _pallas_mm(x, W0) + tx1 + bias


def _bn(h, g, b):
    return h * (g / np.sqrt(1.0 + _EPS_BN)) + b


def kernel(x, edge_index, batch, A, pool_W, pool_b, c1_W0, c1_W1, c1_b,
           bn1_g, bn1_b, c2_W0, c2_W1, c2_b, bn2_g, bn2_b, l1_W, l1_b,
           bn3_g, bn3_b, l2_W, l2_b, bn4_g, bn4_b, l3_W, l3_b):
    src, dst = edge_index[0], edge_index[1]
    n = x.shape[0]
    ab = _pallas_mm(x, jnp.concatenate(
        [pool_W[:_D_IN, :], pool_W[_D_IN:, :]], axis=1))
    raw = ab[src, 0] + ab[dst, 1] + pool_b[0]
    e = _seg_softmax(raw, dst, n) + 0.5
    cluster_j, score, n_merged, n_new = _greedy_merge(e, src, dst, n)
    pair = _sc_take(cluster_j, src) * n_new + _sc_take(cluster_j, dst)
    sp = jnp.sort(pair)
    fmask = jnp.concatenate([jnp.array([True]), sp[1:] != sp[:-1]])
    new_src = jnp.where(fmask, sp // n_new, 0).astype(jnp.int32)
    new_dst = jnp.where(fmask, sp % n_new, 0).astype(jnp.int32)
    x1 = jax.ops.segment_sum(x, cluster_j, num_segments=n) * score[:, None]
    new_batch = jnp.where(jnp.arange(n) < n_new, batch[0],
                          jnp.ones((), batch.dtype))
    h = jax.nn.relu(_bn(_cheb(x1, new_src, new_dst, c1_W0, c1_W1, c1_b, n),
                        bn1_g, bn1_b))
    h = jax.nn.relu(_bn(_cheb(h, new_src, new_dst, c2_W0, c2_W1, c2_b, n),
                        bn2_g, bn2_b))
    gmp = jax.ops.segment_max(h, new_batch, num_segments=1)
    cnt = jax.ops.segment_sum(jnp.ones((h.shape[0], 1), dtype=h.dtype),
                              new_batch, num_segments=1)
    gap = jax.ops.segment_sum(h, new_batch, num_segments=1) / jnp.maximum(cnt, 1.0)
    g = jnp.concatenate([gmp, gap], axis=1)
    h = jax.nn.relu(_bn(g @ l1_W + l1_b, bn3_g, bn3_b))
    h = jax.nn.relu(_bn(h @ l2_W + l2_b, bn4_g, bn4_b))
    feature = h
    out = jax.nn.log_softmax(jax.nn.relu(h @ l3_W + l3_b), axis=-1)
    return (out, feature)
